# baseline (speedup 1.0000x reference)
import functools

import jax
import jax.numpy as jnp
from jax import lax
from jax.experimental import pallas as pl
from jax.experimental.pallas import tpu as pltpu
from jax.experimental.pallas import tpu_sc as plsc

NBUF = 4


def _detile_x(H, B):
    KI = B // 128

    def body(x_ref, y_ref):
        for s in range(8):
            for kk in range(KI):
                y_ref[KI * s + kk, :] = x_ref[s, pl.ds(128 * kk, 128)]

    return pl.pallas_call(
        body,
        grid=(H // 8,),
        in_specs=[pl.BlockSpec((8, B), lambda g: (g, 0))],
        out_specs=pl.BlockSpec((8 * KI, 128), lambda g: (g, 0)),
        out_shape=jax.ShapeDtypeStruct((H * KI, 128), jnp.int32),
    )


def _embed_kernel(B, H, V, D, nc, ns):
    NW = nc * ns
    KI = B // 128
    n_blocks = H * KI
    per_w = n_blocks // NW
    CB = D // 8

    mesh = plsc.VectorSubcoreMesh(core_axis_name="c", subcore_axis_name="s")

    @functools.partial(
        pl.kernel,
        mesh=mesh,
        compiler_params=pltpu.CompilerParams(
            use_tc_tiling_on_sc=False,
            skip_device_barrier=True,
            needs_layout_passes=False,
            disable_bounds_checks=True,
        ),
        out_type=jax.ShapeDtypeStruct((H, CB, KI, 1024), jnp.float32),
        scratch_types=[
            pltpu.VMEM((NBUF, 128), jnp.int32),
            pltpu.VMEM((NBUF, 128, D), jnp.float32),
            pltpu.VMEM((NBUF, D * 128), jnp.float32),
            pltpu.SemaphoreType.DMA,
            pltpu.SemaphoreType.DMA,
            pltpu.SemaphoreType.DMA,
        ],
    )
    def k(y, table_hbm, p5, idx_v, rows_v, tile_v, sem_i, sem_g, sem_o):
        wid = lax.axis_index("s") * nc + lax.axis_index("c")
        t0 = wid * per_w

        iota = lax.iota(jnp.int32, 16)
        r16 = [iota + jnp.int32(16 * g) for g in range(8)]
        pvec = [lax.rem(iota + jnp.int32(d), jnp.int32(16)) for d in range(16)]
        po = [pvec[d] * jnp.int32(128) + iota for d in range(16)]

        def fire_idx(t, b):
            pltpu.async_copy(y.at[t], idx_v.at[b], sem_i)

        def wait_idx(b):
            pltpu.make_async_copy(y.at[0], idx_v.at[b], sem_i).wait()

        def fire_gather(b):
            pltpu.async_copy(table_hbm.at[idx_v.at[b]], rows_v.at[b], sem_g)

        def wait_gather(b):
            pltpu.make_async_copy(
                table_hbm.at[pl.ds(0, 128)], rows_v.at[b], sem_g
            ).wait()

        def transpose(b):
            rb = rows_v.at[b]
            tb = tile_v.at[b]

            @pl.loop(0, D // 16)
            def _(cq):
                c0 = cq * 16
                co = cq * (16 * 128)
                for d in range(16):
                    cols = pvec[d] + c0
                    out0 = po[d] + co
                    for g in range(8):
                        v = plsc.load_gather(rb, [r16[g], cols])
                        plsc.store_scatter(tb, [out0 + jnp.int32(16 * g)], v)

        def fire_store(t, b):
            h = t // KI
            kk = lax.rem(t, KI)
            for cb in range(CB):
                pltpu.async_copy(
                    tile_v.at[b, pl.ds(1024 * cb, 1024)], p5.at[h, cb, kk], sem_o
                )

        def wait_store(b):
            for cb in range(CB):
                pltpu.make_async_copy(
                    tile_v.at[b, pl.ds(1024 * cb, 1024)], p5.at[0, cb, 0], sem_o
                ).wait()

        for b in range(NBUF):
            fire_idx(t0 + b, b)
        wait_idx(0)
        fire_gather(0)
        wait_idx(1)
        fire_gather(1)

        @pl.loop(0, per_w, step=NBUF)
        def _(j0):
            for u in range(NBUF):
                j = j0 + u
                t = t0 + j
                b = u
                b2 = (u + 2) % NBUF
                @pl.when(j + 2 < per_w)
                def _():
                    wait_idx(b2)
                    fire_gather(b2)
                wait_gather(b)
                @pl.when(j >= NBUF)
                def _():
                    wait_store(b)
                transpose(b)
                fire_store(t, b)
                @pl.when(j + NBUF < per_w)
                def _():
                    fire_idx(t + NBUF, b)

        for b in range(NBUF):
            wait_store(b)

    return k


def kernel(x, table):
    B, H = x.shape
    V, D = table.shape
    y = _detile_x(H, B)(jnp.transpose(x))
    info = plsc.get_sparse_core_info()
    p5 = _embed_kernel(B, H, V, D, info.num_cores, info.num_subcores)(y, table)
    o = jnp.transpose(p5.reshape(H, D // 8, B // 128, 8, 128), (2, 4, 0, 1, 3))
    return o.reshape(B, H, D)


# speedup vs baseline: 1.2410x; 1.2410x over previous
import functools

import jax
import jax.numpy as jnp
from jax import lax
from jax.experimental import pallas as pl
from jax.experimental.pallas import tpu as pltpu
from jax.experimental.pallas import tpu_sc as plsc

CHUNK = 256
IDXW = 128
GPC = CHUNK // IDXW
GBUF = 4
TBUF = 2


def _detile_x(H, B):
    KI = B // 128

    def body(x_ref, y_ref):
        for s in range(8):
            for kk in range(KI):
                y_ref[KI * s + kk, :] = x_ref[s, pl.ds(128 * kk, 128)]

    return pl.pallas_call(
        body,
        grid=(H // 8,),
        in_specs=[pl.BlockSpec((8, B), lambda g: (g, 0))],
        out_specs=pl.BlockSpec((8 * KI, 128), lambda g: (g, 0)),
        out_shape=jax.ShapeDtypeStruct((H * KI, 128), jnp.int32),
    )


def _gather_call(n, dim, nc, ns):
    NW = nc * ns
    per_w = n // NW
    n_chunks = per_w // CHUNK
    assert n_chunks % GBUF == 0
    idx_rows_per_w = per_w // IDXW

    mesh = plsc.VectorSubcoreMesh(core_axis_name="c", subcore_axis_name="s")

    @functools.partial(
        pl.kernel,
        mesh=mesh,
        compiler_params=pltpu.CompilerParams(
            use_tc_tiling_on_sc=False, skip_device_barrier=True
        ),
        out_type=jax.ShapeDtypeStruct((n, dim), jnp.float32),
        scratch_types=[
            pltpu.VMEM((GBUF, GPC, IDXW), jnp.int32),
            pltpu.VMEM((GBUF, CHUNK, dim), jnp.float32),
            pltpu.SemaphoreType.DMA,
            pltpu.SemaphoreType.DMA,
            pltpu.SemaphoreType.DMA,
        ],
    )
    def k(idx_hbm, table_hbm, out_hbm, idx_v, rows_v, sem_i, sem_g, sem_o):
        wid = lax.axis_index("s") * nc + lax.axis_index("c")
        row_base = wid * idx_rows_per_w
        out_base = wid * per_w

        def fire_idx(g, b):
            pltpu.async_copy(
                idx_hbm.at[pl.ds(row_base + g * GPC, GPC)], idx_v.at[b], sem_i
            )

        def wait_idx(b):
            pltpu.make_async_copy(
                idx_hbm.at[pl.ds(row_base, GPC)], idx_v.at[b], sem_i
            ).wait()

        def fire_gathers(b):
            for j in range(GPC):
                pltpu.async_copy(
                    table_hbm.at[idx_v.at[b].at[j]],
                    rows_v.at[b].at[pl.ds(j * IDXW, IDXW)],
                    sem_g,
                )

        def wait_gathers(b):
            pltpu.make_async_copy(
                out_hbm.at[pl.ds(out_base, CHUNK)], rows_v.at[b], sem_g
            ).wait()

        def fire_store(g, b):
            pltpu.async_copy(
                rows_v.at[b], out_hbm.at[pl.ds(out_base + g * CHUNK, CHUNK)], sem_o
            )

        def wait_store(b):
            pltpu.make_async_copy(
                rows_v.at[b], out_hbm.at[pl.ds(out_base, CHUNK)], sem_o
            ).wait()

        for b in range(GBUF):
            fire_idx(b, b)
        wait_idx(0)
        fire_gathers(0)

        @pl.loop(0, n_chunks, step=GBUF)
        def _(g0):
            for b in range(GBUF):
                g = g0 + b
                b1 = (b + 1) % GBUF
                @pl.when(g + 1 < n_chunks)
                def _():
                    @pl.when(g + 1 >= GBUF)
                    def _():
                        wait_store(b1)
                    wait_idx(b1)
                    fire_gathers(b1)
                wait_gathers(b)
                fire_store(g, b)
                @pl.when(g + GBUF < n_chunks)
                def _():
                    fire_idx(g + GBUF, b)

        for b in range(GBUF):
            wait_store(b)

    return k


def _transpose_call(B, H, D, nc, ns):
    NW = nc * ns
    KI = B // 128
    n_blocks = H * KI
    per_w = n_blocks // NW
    assert per_w % TBUF == 0
    CB = D // 8

    mesh = plsc.VectorSubcoreMesh(core_axis_name="c", subcore_axis_name="s")

    @functools.partial(
        pl.kernel,
        mesh=mesh,
        compiler_params=pltpu.CompilerParams(
            use_tc_tiling_on_sc=False,
            skip_device_barrier=True,
            needs_layout_passes=False,
            disable_bounds_checks=True,
        ),
        out_type=jax.ShapeDtypeStruct((H, CB, KI, 1024), jnp.float32),
        scratch_types=[
            pltpu.VMEM((TBUF, 128, D), jnp.float32),
            pltpu.VMEM((TBUF, D * 128), jnp.float32),
            pltpu.SemaphoreType.DMA,
            pltpu.SemaphoreType.DMA,
        ],
    )
    def k(q, p5, rows_v, tile_v, sem_g, sem_o):
        wid = lax.axis_index("s") * nc + lax.axis_index("c")
        t0 = wid * per_w

        iota = lax.iota(jnp.int32, 16)
        r16 = [iota + jnp.int32(16 * g) for g in range(8)]
        pvec = [lax.rem(iota + jnp.int32(d), jnp.int32(16)) for d in range(16)]
        po = [pvec[d] * jnp.int32(128) + iota for d in range(16)]

        def fire_load(t, b):
            pltpu.async_copy(q.at[pl.ds(t * 128, 128)], rows_v.at[b], sem_g)

        def wait_load(b):
            pltpu.make_async_copy(
                q.at[pl.ds(0, 128)], rows_v.at[b], sem_g
            ).wait()

        def transpose(b):
            rb = rows_v.at[b]
            tb = tile_v.at[b]

            @pl.loop(0, D // 16)
            def _(cq):
                c0 = cq * 16
                co = cq * (16 * 128)
                for d in range(16):
                    cols = pvec[d] + c0
                    out0 = po[d] + co
                    vs = [
                        plsc.load_gather(rb, [r16[g], cols]) for g in range(8)
                    ]
                    for g in range(8):
                        plsc.store_scatter(tb, [out0 + jnp.int32(16 * g)], vs[g])

        def fire_store(t, b):
            h = t // KI
            kk = lax.rem(t, KI)
            for cb in range(CB):
                pltpu.async_copy(
                    tile_v.at[b, pl.ds(1024 * cb, 1024)], p5.at[h, cb, kk], sem_o
                )

        def wait_store(b):
            for cb in range(CB):
                pltpu.make_async_copy(
                    tile_v.at[b, pl.ds(1024 * cb, 1024)], p5.at[0, cb, 0], sem_o
                ).wait()

        fire_load(t0, 0)
        fire_load(t0 + 1, 1)

        @pl.loop(0, per_w, step=TBUF)
        def _(j0):
            for u in range(TBUF):
                j = j0 + u
                t = t0 + j
                b = u
                wait_load(b)
                @pl.when(j >= TBUF)
                def _():
                    wait_store(b)
                transpose(b)
                fire_store(t, b)
                @pl.when(j + TBUF < per_w)
                def _():
                    fire_load(t + TBUF, b)

        for b in range(TBUF):
            wait_store(b)

    return k


def kernel(x, table):
    B, H = x.shape
    V, D = table.shape
    n = B * H
    y = _detile_x(H, B)(jnp.transpose(x))
    info = plsc.get_sparse_core_info()
    nc, ns = info.num_cores, info.num_subcores
    q = _gather_call(n, D, nc, ns)(y, table)
    p5 = _transpose_call(B, H, D, nc, ns)(q)
    o = jnp.transpose(p5.reshape(H, D // 8, B // 128, 8, 128), (2, 4, 0, 1, 3))
    return o.reshape(B, H, D)


# speedup vs baseline: 1.4851x; 1.1968x over previous
import functools

import jax
import jax.numpy as jnp
from jax import lax
from jax.experimental import pallas as pl
from jax.experimental.pallas import tpu as pltpu
from jax.experimental.pallas import tpu_sc as plsc

NBUF = 4


def _detile_x(H, B):
    KI = B // 128

    def body(x_ref, y_ref):
        for s in range(8):
            for kk in range(KI):
                y_ref[KI * s + kk, :] = x_ref[s, pl.ds(128 * kk, 128)]

    return pl.pallas_call(
        body,
        grid=(H // 8,),
        in_specs=[pl.BlockSpec((8, B), lambda g: (g, 0))],
        out_specs=pl.BlockSpec((8 * KI, 128), lambda g: (g, 0)),
        out_shape=jax.ShapeDtypeStruct((H * KI, 128), jnp.int32),
    )


def _embed_fused(B, H, V, D, nc, ns):
    NW = nc * ns
    KI = B // 128
    n_blocks = H * KI
    per_w = n_blocks // NW
    assert per_w % NBUF == 0
    CB = D // 8

    mesh = plsc.VectorSubcoreMesh(core_axis_name="c", subcore_axis_name="s")

    @functools.partial(
        pl.kernel,
        mesh=mesh,
        compiler_params=pltpu.CompilerParams(
            use_tc_tiling_on_sc=True,
            skip_device_barrier=True,
            needs_layout_passes=False,
            disable_bounds_checks=True,
        ),
        out_type=jax.ShapeDtypeStruct((H, CB, KI, 8, 128), jnp.float32),
        scratch_types=[
            pltpu.VMEM((NBUF, 128), jnp.int32),
            pltpu.VMEM((NBUF, 128), jnp.int32),
            pltpu.VMEM((NBUF, 128, 128), jnp.float32),
            pltpu.VMEM((NBUF, D, 128), jnp.float32),
            pltpu.SemaphoreType.DMA,
            pltpu.SemaphoreType.DMA,
            pltpu.SemaphoreType.DMA,
        ],
    )
    def k(y, table2, p5, idx_v, idh_v, prs_v, tile_v, sem_i, sem_g, sem_o):
        wid = lax.axis_index("s") * nc + lax.axis_index("c")
        t0 = wid * per_w

        iota = lax.iota(jnp.int32, 16)
        r16 = [iota + jnp.int32(16 * g) for g in range(8)]
        pvec = [lax.rem(iota + jnp.int32(d), jnp.int32(16)) for d in range(16)]

        def fire_idx(t, b):
            pltpu.async_copy(y.at[t], idx_v.at[b], sem_i)

        def wait_idx(b):
            pltpu.make_async_copy(y.at[0], idx_v.at[b], sem_i).wait()

        def shift_idx(b):
            for g in range(8):
                idh_v[b, pl.ds(16 * g, 16)] = (
                    idx_v[b, pl.ds(16 * g, 16)] >> jnp.int32(1)
                )

        def fire_gather(b):
            pltpu.async_copy(table2.at[idh_v.at[b]], prs_v.at[b], sem_g)

        def wait_gather(b):
            pltpu.make_async_copy(
                table2.at[pl.ds(0, 128)], prs_v.at[b], sem_g
            ).wait()

        def transpose(b):
            rb = prs_v.at[b]
            tb = tile_v.at[b]
            off = [
                (idx_v[b, pl.ds(16 * g, 16)] & jnp.int32(1)) << jnp.int32(6)
                for g in range(8)
            ]

            @pl.loop(0, D // 16)
            def _(cq):
                c0 = cq * 16
                for d in range(16):
                    cols = pvec[d] + c0
                    vs = [
                        plsc.load_gather(rb, [r16[g], cols + off[g]])
                        for g in range(8)
                    ]
                    for g in range(8):
                        plsc.store_scatter(tb, [cols, r16[g]], vs[g])

        def fire_store(t, b):
            h = t // KI
            kk = lax.rem(t, KI)
            for cb in range(CB):
                pltpu.async_copy(
                    tile_v.at[b, pl.ds(8 * cb, 8)], p5.at[h, cb, kk], sem_o
                )

        def wait_store(b):
            for cb in range(CB):
                pltpu.make_async_copy(
                    tile_v.at[b, pl.ds(8 * cb, 8)], p5.at[0, cb, 0], sem_o
                ).wait()

        for b in range(NBUF):
            fire_idx(t0 + b, b)
        for b in range(2):
            wait_idx(b)
            shift_idx(b)
            fire_gather(b)

        @pl.loop(0, per_w, step=NBUF)
        def _(j0):
            for u in range(NBUF):
                j = j0 + u
                t = t0 + j
                b = u
                b2 = (u + 2) % NBUF
                @pl.when(j + 2 < per_w)
                def _():
                    wait_idx(b2)
                    shift_idx(b2)
                    fire_gather(b2)
                wait_gather(b)
                @pl.when(j >= NBUF)
                def _():
                    wait_store(b)
                transpose(b)
                fire_store(t, b)
                @pl.when(j + NBUF < per_w)
                def _():
                    fire_idx(t + NBUF, b)

        for b in range(NBUF):
            wait_store(b)

    return k


def kernel(x, table):
    B, H = x.shape
    V, D = table.shape
    y = _detile_x(H, B)(jnp.transpose(x))
    table2 = table.reshape(V // 2, 2 * D)
    info = plsc.get_sparse_core_info()
    p5 = _embed_fused(B, H, V, D, info.num_cores, info.num_subcores)(y, table2)
    o = jnp.transpose(p5, (2, 4, 0, 1, 3))
    return o.reshape(B, H, D)
